# baseline (speedup 1.0000x reference)
import functools

import jax
import jax.numpy as jnp
from jax import lax
from jax.experimental import pallas as pl
from jax.experimental.pallas import tpu as pltpu
from jax.experimental.pallas import tpu_sc as plsc

_NP = 20000
_NPP = 20480
_NCHK = 10
_CW = 2048
_NT = 1000
_NTP = 1024
_TPW = 32

_BIG = float(2.0 ** 30)


def _sc_call(preds_c, tgt_w):
    mesh = plsc.VectorSubcoreMesh(core_axis_name="c", subcore_axis_name="s")

    @functools.partial(
        pl.kernel,
        mesh=mesh,
        out_type=[jax.ShapeDtypeStruct((_NTP, 16), jnp.float32)] * 4,
        scratch_types=[
            pltpu.VMEM((8, _CW), jnp.float32),
            pltpu.VMEM((_CW,), jnp.float32),
            pltpu.VMEM((_CW,), jnp.float32),
            pltpu.VMEM((8 * _TPW, 16), jnp.float32),
            pltpu.VMEM((_TPW, 16), jnp.float32),
            pltpu.VMEM((_TPW, 16), jnp.float32),
            pltpu.VMEM((_TPW, 16), jnp.float32),
            pltpu.VMEM((_TPW, 16), jnp.float32),
        ],
    )
    def sc(phbm, thbm, obv, obk, obj, obl,
           pbuf, keyv, vpv, tvm, cbv, cbk, cbj, cbl):
        wid = lax.axis_index("c") * 16 + lax.axis_index("s")
        pltpu.sync_copy(thbm.at[wid], tvm)

        def initloop(t, u):
            cbv[t] = jnp.full((16,), -jnp.inf, jnp.float32)
            cbk[t] = jnp.full((16,), jnp.inf, jnp.float32)
            cbj[t] = jnp.full((16,), _BIG, jnp.float32)
            cbl[t] = jnp.full((16,), 0.0, jnp.float32)
            return u
        lax.fori_loop(0, _TPW, initloop, 0)

        iota = lax.iota(jnp.int32, 16)

        def chunk_body(c, u):
            pltpu.sync_copy(phbm.at[c], pbuf)

            def prep(i, v):
                s = i * 16
                sc_ = pbuf[6, pl.ds(s, 16)]
                valid = sc_ > 0.5
                keyv[pl.ds(s, 16)] = jnp.where(valid, sc_, jnp.inf)
                vp = (((pbuf[3, pl.ds(s, 16)] - pbuf[0, pl.ds(s, 16)] + 1.0)
                       * (pbuf[4, pl.ds(s, 16)] - pbuf[1, pl.ds(s, 16)] + 1.0))
                      * (pbuf[5, pl.ds(s, 16)] - pbuf[2, pl.ds(s, 16)] + 1.0))
                vpv[pl.ds(s, 16)] = jnp.where(valid, vp, jnp.inf)
                return v
            lax.fori_loop(0, _CW // 16, prep, 0)

            def tloop(t, u2):
                tf = [tvm[t + f * _TPW] for f in range(6)]
                ttl = tf[:3]
                tbr = tf[3:]
                vt = ((tbr[0] - ttl[0] + 1.0) * (tbr[1] - ttl[1] + 1.0)
                      * (tbr[2] - ttl[2] + 1.0))

                def inner(i, carry):
                    bv, bk, bj, bl = carry
                    s = i * 16
                    p = [pbuf[f, pl.ds(s, 16)] for f in range(6)]
                    key = keyv[pl.ds(s, 16)]
                    vp = vpv[pl.ds(s, 16)]
                    lb = pbuf[7, pl.ds(s, 16)]
                    jv = (iota + (c * _CW + s)).astype(jnp.float32)

                    e0 = (jnp.minimum(tbr[0], p[3])
                          - jnp.maximum(ttl[0], p[0]) + 1.0)
                    e1 = (jnp.minimum(tbr[1], p[4])
                          - jnp.maximum(ttl[1], p[1]) + 1.0)
                    e2 = (jnp.minimum(tbr[2], p[5])
                          - jnp.maximum(ttl[2], p[2]) + 1.0)
                    inter = (e0 * e1) * e2
                    union = (vt + vp) - inter
                    ov1 = ((tbr[0] > p[0]) | (tbr[1] > p[1])
                           | (tbr[2] > p[2]))
                    ov2 = ((ttl[0] < p[3]) | (ttl[1] < p[4])
                           | (ttl[2] < p[5]))
                    val = jnp.where(ov1 & ov2, inter / union, 0.0)

                    upd = (val > bv) | ((val == bv) & (key < bk))
                    return (jnp.where(upd, val, bv),
                            jnp.where(upd, key, bk),
                            jnp.where(upd, jv, bj),
                            jnp.where(upd, lb, bl))

                bv, bk, bj, bl = lax.fori_loop(
                    0, _CW // 16, inner, (cbv[t], cbk[t], cbj[t], cbl[t]))
                cbv[t] = bv
                cbk[t] = bk
                cbj[t] = bj
                cbl[t] = bl
                return u2
            lax.fori_loop(0, _TPW, tloop, 0)
            return u
        lax.fori_loop(0, _NCHK, chunk_body, 0)

        base = wid * _TPW
        pltpu.sync_copy(cbv, obv.at[pl.ds(base, _TPW)])
        pltpu.sync_copy(cbk, obk.at[pl.ds(base, _TPW)])
        pltpu.sync_copy(cbj, obj.at[pl.ds(base, _TPW)])
        pltpu.sync_copy(cbl, obl.at[pl.ds(base, _TPW)])

    return sc(preds_c, tgt_w)


def _merge_body(bvr, bkr, bjr, blr, iou_ref, lab_ref):
    bv = bvr[...]
    bk = bkr[...]
    bj = bjr[...]
    bl = blr[...]
    bm = jnp.max(bv, axis=1, keepdims=True)
    tie = bv == bm
    km = jnp.min(jnp.where(tie, bk, jnp.inf), axis=1, keepdims=True)
    tie2 = tie & (bk == km)
    jm = jnp.min(jnp.where(tie2, bj, _BIG), axis=1, keepdims=True)
    tie3 = tie2 & (bj == jm)
    lab = jnp.max(jnp.where(tie3, bl, -jnp.inf), axis=1, keepdims=True)
    iou_ref[...] = bm
    lab_ref[...] = lab


@jax.jit
def _run_sc(preds_c, tgt_w):
    bv, bk, bj, bl = _sc_call(preds_c, tgt_w)
    iou, lab = pl.pallas_call(
        _merge_body,
        out_shape=[jax.ShapeDtypeStruct((_NTP, 1), jnp.float32)] * 2,
    )(bv, bk, bj, bl)
    return iou, lab


def kernel(pred_boxes, pred_scores, pred_labels, target_boxes, target_labels):
    preds = jnp.concatenate(
        [pred_boxes, pred_scores[:, None], pred_labels[:, None]], axis=1).T
    preds = jnp.pad(preds, ((0, 0), (0, _NPP - _NP)))
    preds_c = preds.reshape(8, _NCHK, _CW).transpose(1, 0, 2)
    tgt = jnp.concatenate(
        [target_boxes, jnp.zeros((_NT, 2), jnp.float32)], axis=1)
    tgt = jnp.pad(tgt, ((0, _NTP - _NT), (0, 0)))
    tgt_w = tgt.reshape(32, _TPW, 8).transpose(0, 2, 1).reshape(32, 8 * _TPW)
    tgt_w = jnp.broadcast_to(tgt_w[:, :, None], (32, 8 * _TPW, 16))
    iou, lab = _run_sc(preds_c, tgt_w)
    true_ious = iou.reshape(_NTP)[:_NT]
    pcp_best = lab.reshape(_NTP)[:_NT]
    hit = true_ious > 0.5
    return true_ious, pcp_best, hit, target_labels


# speedup vs baseline: 3.0410x; 3.0410x over previous
import functools

import jax
import jax.numpy as jnp
from jax import lax
from jax.experimental import pallas as pl
from jax.experimental.pallas import tpu as pltpu
from jax.experimental.pallas import tpu_sc as plsc

_NP = 20000
_NPP = 20480
_SPLIT = 16384
_CH = 256
_NCH = _SPLIT // _CH
_NT = 1000
_NB = 8
_NTP = 1024
_TB = 8
_NBLK = _NTP // (_NB * _TB)

_CW = 2048
_SC_NCHK = (_NPP - _SPLIT) // _CW
_TPW = 32

_BIG = float(2.0 ** 30)


def _tfields(t):
    ttl = [t[:, d:d + 1] for d in range(3)]
    tbr = [t[:, 3 + d:4 + d] for d in range(3)]
    vt = ((tbr[0] - ttl[0] + 1.0) * (tbr[1] - ttl[1] + 1.0)
          * (tbr[2] - ttl[2] + 1.0))
    return ttl, tbr, vt


def _pairval(ttl, tbr, vt, p, vp):
    e0 = jnp.minimum(tbr[0], p[3]) - jnp.maximum(ttl[0], p[0]) + 1.0
    e1 = jnp.minimum(tbr[1], p[4]) - jnp.maximum(ttl[1], p[1]) + 1.0
    e2 = jnp.minimum(tbr[2], p[5]) - jnp.maximum(ttl[2], p[2]) + 1.0
    inter = (e0 * e1) * e2
    union = (vt + vp) - inter
    ov1 = (tbr[0] > p[0]) | (tbr[1] > p[1]) | (tbr[2] > p[2])
    ov2 = (ttl[0] < p[3]) | (ttl[1] < p[4]) | (ttl[2] < p[5])
    return jnp.where(ov1 & ov2, inter / union, 0.0)


def _merge(carry, val, key, jv, lb):
    bv, bk, bj, bl = carry
    upd = (val > bv) | ((val == bv) & (key < bk))
    return (jnp.where(upd, val, bv), jnp.where(upd, key, bk),
            jnp.where(upd, jv, bj), jnp.where(upd, lb, bl))


def _lexreduce(bv, bk, bj, bl):
    bm = jnp.max(bv, axis=1, keepdims=True)
    tie = bv == bm
    km = jnp.min(jnp.where(tie, bk, jnp.inf), axis=1, keepdims=True)
    tie2 = tie & (bk == km)
    jm = jnp.min(jnp.where(tie2, bj, _BIG), axis=1, keepdims=True)
    tie3 = tie2 & (bj == jm)
    lab = jnp.max(jnp.where(tie3, bl, -jnp.inf), axis=1, keepdims=True)
    return bm, km, jm, lab


def _tc_body(tref, pref, v_ref, k_ref, j_ref, l_ref, scr):
    @pl.when(pl.program_id(0) == 0)
    def _init():
        score = pref[6]
        valid = score > 0.5
        scr[0:8, :] = jnp.where(valid, score, jnp.inf)
        vp = (((pref[3] - pref[0] + 1.0) * (pref[4] - pref[1] + 1.0))
              * (pref[5] - pref[2] + 1.0))
        scr[8:16, :] = jnp.where(valid, vp, jnp.inf)
        scr[16:24, :] = jax.lax.broadcasted_iota(
            jnp.int32, (8, _SPLIT), 1).astype(jnp.float32)

    tf = [_tfields(tref[b]) for b in range(_NB)]

    def chunk(c, carry):
        s = c * _CH
        p = [pref[f, :, pl.ds(s, _CH)] for f in range(6)]
        key = scr[0:8, pl.ds(s, _CH)]
        vp = scr[8:16, pl.ds(s, _CH)]
        jv = scr[16:24, pl.ds(s, _CH)]
        lb = pref[7, :, pl.ds(s, _CH)]
        return tuple(
            _merge(carry[b], _pairval(*tf[b], p, vp), key, jv, lb)
            for b in range(_NB))

    full = functools.partial(jnp.full, (_TB, _CH), dtype=jnp.float32)
    init = (full(-jnp.inf), full(jnp.inf), full(_BIG), full(0.0))
    cs = lax.fori_loop(0, _NCH, chunk, (init,) * _NB, unroll=16)

    for b in range(_NB):
        bm, km, jm, lab = _lexreduce(*cs[b])
        v_ref[b] = bm
        k_ref[b] = km
        j_ref[b] = jm
        l_ref[b] = lab


def _tc_call(tgt, preds):
    return pl.pallas_call(
        _tc_body,
        grid=(_NBLK,),
        in_specs=[
            pl.BlockSpec((_NB, _TB, 8), lambda i: (i, 0, 0)),
            pl.BlockSpec((8, 8, _SPLIT), lambda i: (0, 0, 0)),
        ],
        out_specs=[pl.BlockSpec((_NB, _TB, 1), lambda i: (i, 0, 0))] * 4,
        out_shape=[
            jax.ShapeDtypeStruct((_NB * _NBLK, _TB, 1), jnp.float32)] * 4,
        scratch_shapes=[pltpu.VMEM((24, _SPLIT), jnp.float32)],
    )(tgt, preds)


def _sc_call(preds_c, tgt_w):
    mesh = plsc.VectorSubcoreMesh(core_axis_name="c", subcore_axis_name="s")

    @functools.partial(
        pl.kernel,
        mesh=mesh,
        out_type=[jax.ShapeDtypeStruct((_NTP, 16), jnp.float32)] * 4,
        scratch_types=[
            pltpu.VMEM((8, _CW), jnp.float32),
            pltpu.VMEM((_CW,), jnp.float32),
            pltpu.VMEM((_CW,), jnp.float32),
            pltpu.VMEM((8 * _TPW, 16), jnp.float32),
            pltpu.VMEM((_TPW, 16), jnp.float32),
            pltpu.VMEM((_TPW, 16), jnp.float32),
            pltpu.VMEM((_TPW, 16), jnp.float32),
            pltpu.VMEM((_TPW, 16), jnp.float32),
        ],
    )
    def sc(phbm, thbm, obv, obk, obj, obl,
           pbuf, keyv, vpv, tvm, cbv, cbk, cbj, cbl):
        wid = lax.axis_index("c") * 16 + lax.axis_index("s")
        pltpu.sync_copy(thbm.at[wid], tvm)

        def initloop(t, u):
            cbv[t] = jnp.full((16,), -jnp.inf, jnp.float32)
            cbk[t] = jnp.full((16,), jnp.inf, jnp.float32)
            cbj[t] = jnp.full((16,), _BIG, jnp.float32)
            cbl[t] = jnp.full((16,), 0.0, jnp.float32)
            return u
        lax.fori_loop(0, _TPW, initloop, 0)

        iota = lax.iota(jnp.int32, 16)

        def chunk_body(c, u):
            pltpu.sync_copy(phbm.at[c], pbuf)

            def prep(i, v):
                s = i * 16
                sc_ = pbuf[6, pl.ds(s, 16)]
                valid = sc_ > 0.5
                keyv[pl.ds(s, 16)] = jnp.where(valid, sc_, jnp.inf)
                vp = (((pbuf[3, pl.ds(s, 16)] - pbuf[0, pl.ds(s, 16)] + 1.0)
                       * (pbuf[4, pl.ds(s, 16)] - pbuf[1, pl.ds(s, 16)] + 1.0))
                      * (pbuf[5, pl.ds(s, 16)] - pbuf[2, pl.ds(s, 16)] + 1.0))
                vpv[pl.ds(s, 16)] = jnp.where(valid, vp, jnp.inf)
                return v
            lax.fori_loop(0, _CW // 16, prep, 0)

            def tloop(t, u2):
                tf = [tvm[t + f * _TPW] for f in range(6)]
                ttl = tf[:3]
                tbr = tf[3:]
                vt = ((tbr[0] - ttl[0] + 1.0) * (tbr[1] - ttl[1] + 1.0)
                      * (tbr[2] - ttl[2] + 1.0))

                def inner(i, carry):
                    bv, bk, bj, bl = carry
                    s = i * 16
                    p = [pbuf[f, pl.ds(s, 16)] for f in range(6)]
                    key = keyv[pl.ds(s, 16)]
                    vp = vpv[pl.ds(s, 16)]
                    lb = pbuf[7, pl.ds(s, 16)]
                    jv = (iota + (_SPLIT + c * _CW + s)).astype(jnp.float32)

                    e0 = (jnp.minimum(tbr[0], p[3])
                          - jnp.maximum(ttl[0], p[0]) + 1.0)
                    e1 = (jnp.minimum(tbr[1], p[4])
                          - jnp.maximum(ttl[1], p[1]) + 1.0)
                    e2 = (jnp.minimum(tbr[2], p[5])
                          - jnp.maximum(ttl[2], p[2]) + 1.0)
                    inter = (e0 * e1) * e2
                    union = (vt + vp) - inter
                    ov1 = ((tbr[0] > p[0]) | (tbr[1] > p[1])
                           | (tbr[2] > p[2]))
                    ov2 = ((ttl[0] < p[3]) | (ttl[1] < p[4])
                           | (ttl[2] < p[5]))
                    val = jnp.where(ov1 & ov2, inter / union, 0.0)

                    upd = (val > bv) | ((val == bv) & (key < bk))
                    return (jnp.where(upd, val, bv),
                            jnp.where(upd, key, bk),
                            jnp.where(upd, jv, bj),
                            jnp.where(upd, lb, bl))

                bv, bk, bj, bl = lax.fori_loop(
                    0, _CW // 16, inner, (cbv[t], cbk[t], cbj[t], cbl[t]))
                cbv[t] = bv
                cbk[t] = bk
                cbj[t] = bj
                cbl[t] = bl
                return u2
            lax.fori_loop(0, _TPW, tloop, 0)
            return u
        lax.fori_loop(0, _SC_NCHK, chunk_body, 0)

        base = wid * _TPW
        pltpu.sync_copy(cbv, obv.at[pl.ds(base, _TPW)])
        pltpu.sync_copy(cbk, obk.at[pl.ds(base, _TPW)])
        pltpu.sync_copy(cbj, obj.at[pl.ds(base, _TPW)])
        pltpu.sync_copy(cbl, obl.at[pl.ds(base, _TPW)])

    return sc(preds_c, tgt_w)


def _merge_body(tv, tk, tj, tl, sv, sk, sj, sl, iou_ref, lab_ref):
    bv = jnp.concatenate([tv[...], sv[...]], axis=1)
    bk = jnp.concatenate([tk[...], sk[...]], axis=1)
    bj = jnp.concatenate([tj[...], sj[...]], axis=1)
    bl = jnp.concatenate([tl[...], sl[...]], axis=1)
    bm, _, _, lab = _lexreduce(bv, bk, bj, bl)
    iou_ref[...] = bm
    lab_ref[...] = lab


@jax.jit
def _run(tgt, preds_tc, preds_sc, tgt_w):
    sv, sk, sj, sl = _sc_call(preds_sc, tgt_w)
    tv, tk, tj, tl = _tc_call(tgt, preds_tc)
    iou, lab = pl.pallas_call(
        _merge_body,
        out_shape=[jax.ShapeDtypeStruct((_NTP, 1), jnp.float32)] * 2,
    )(tv.reshape(_NTP, 1), tk.reshape(_NTP, 1), tj.reshape(_NTP, 1),
      tl.reshape(_NTP, 1), sv, sk, sj, sl)
    return iou, lab


def kernel(pred_boxes, pred_scores, pred_labels, target_boxes, target_labels):
    preds = jnp.concatenate(
        [pred_boxes, pred_scores[:, None], pred_labels[:, None]], axis=1).T
    preds = jnp.pad(preds, ((0, 0), (0, _NPP - _NP)))
    preds_tc = jnp.broadcast_to(
        preds[:, None, :_SPLIT], (8, 8, _SPLIT))
    preds_sc = (preds[:, _SPLIT:]
                .reshape(8, _SC_NCHK, _CW).transpose(1, 0, 2))
    tgt = jnp.concatenate(
        [target_boxes, jnp.zeros((_NT, 2), jnp.float32)], axis=1)
    tgt = jnp.pad(tgt, ((0, _NTP - _NT), (0, 0)))
    tgt_w = tgt.reshape(32, _TPW, 8).transpose(0, 2, 1).reshape(32, 8 * _TPW)
    tgt_w = jnp.broadcast_to(tgt_w[:, :, None], (32, 8 * _TPW, 16))
    tgt_b = tgt.reshape(_NB * _NBLK, _TB, 8)
    iou, lab = _run(tgt_b, preds_tc, preds_sc, tgt_w)
    true_ious = iou.reshape(_NTP)[:_NT]
    pcp_best = lab.reshape(_NTP)[:_NT]
    hit = true_ious > 0.5
    return true_ious, pcp_best, hit, target_labels
